# baseline (speedup 1.0000x reference)
import functools

import jax
import jax.numpy as jnp
from jax import lax
from jax.experimental import pallas as pl
from jax.experimental.pallas import tpu as pltpu
from jax.experimental.pallas import tpu_sc as plsc

N = 10000
E = 320000
D = 128
NPAD = 10240
CHUNK = 128
NC = 2
NS = 16
NW = NC * NS
CPW = 80
NCHP = CPW * NW
EPAD = NCHP * CHUNK
RPW = NPAD // NS
NBUF = 4
T_ITERS = CPW // NBUF
DH = D // 2

_SC_MESH = dict(
    mesh=plsc.VectorSubcoreMesh(core_axis_name="c", subcore_axis_name="s"),
    compiler_params=pltpu.CompilerParams(needs_layout_passes=False),
)


def _deg_body(src_hbm, dst_hbm, out_hbm, srcall, dstall, ones_v, zeros_v,
              dsrc_sh, ddst_sh, sem):
  cid = lax.axis_index("c")
  sid = lax.axis_index("s")
  w = cid * NS + sid

  pltpu.sync_copy(src_hbm.at[w], srcall)
  pltpu.sync_copy(dst_hbm.at[w], dstall)

  def initz(i, _):
    zeros_v[pl.ds(i * 16, 16)] = jnp.zeros((16,), jnp.float32)
    return 0
  lax.fori_loop(0, RPW // 16, initz, 0)

  def inito(i, _):
    ones_v[pl.ds(i * 16, 16)] = jnp.ones((16,), jnp.float32)
    return 0
  lax.fori_loop(0, CHUNK // 16, inito, 0)

  pltpu.sync_copy(zeros_v, dsrc_sh.at[pl.ds(sid * RPW, RPW)])
  pltpu.sync_copy(zeros_v, ddst_sh.at[pl.ds(sid * RPW, RPW)])
  plsc.subcore_barrier()

  def chunk_body(t, _):
    d1 = pltpu.async_copy(ones_v, dsrc_sh.at[srcall.at[t]], sem, add=True)
    d2 = pltpu.async_copy(ones_v, ddst_sh.at[dstall.at[t]], sem, add=True)

    @pl.when(t >= 8)
    def _():
      d1.wait()
      d2.wait()
    return 0
  lax.fori_loop(0, CPW, chunk_body, 0)

  for _ in range(8):
    pltpu.make_async_copy(ones_v, dsrc_sh.at[srcall.at[0]], sem).wait()
    pltpu.make_async_copy(ones_v, ddst_sh.at[dstall.at[0]], sem).wait()

  plsc.subcore_barrier()
  sl = pl.ds(sid * RPW, RPW)
  pltpu.sync_copy(dsrc_sh.at[sl], out_hbm.at[0, cid, sl])
  pltpu.sync_copy(ddst_sh.at[sl], out_hbm.at[1, cid, sl])


def _degrees(src2d, dst2d):
  fn = pl.kernel(
      _deg_body,
      out_type=jax.ShapeDtypeStruct((2, NC, NPAD), jnp.float32),
      scratch_types=[
          pltpu.VMEM((CPW, CHUNK), jnp.int32),
          pltpu.VMEM((CPW, CHUNK), jnp.int32),
          pltpu.VMEM((CHUNK,), jnp.float32),
          pltpu.VMEM((RPW,), jnp.float32),
          pltpu.VMEM_SHARED((NPAD,), jnp.float32),
          pltpu.VMEM_SHARED((NPAD,), jnp.float32),
          pltpu.SemaphoreType.DMA,
      ],
      **_SC_MESH,
  )
  return fn(src2d, dst2d)


def _scale_body(d_ref, s_ref):
  d = d_ref[...]
  s_ref[...] = lax.rsqrt(jnp.maximum(d[:, 0] + d[:, 1], 1.0))


def _scales(deg_parts):
  d4 = deg_parts.reshape(2, NC, NPAD // 128, 128)
  s = pl.pallas_call(
      _scale_body,
      out_shape=jax.ShapeDtypeStruct((2, NPAD // 128, 128), jnp.float32),
  )(d4)
  return s.reshape(2, NPAD)


def _wcalc_body(src_hbm, dst_hbm, ef_hbm, sout_hbm, sin_hbm, w_hbm,
                srcall, dstall, efall, sout_v, sin_v):
  cid = lax.axis_index("c")
  sid = lax.axis_index("s")
  w = cid * NS + sid

  pltpu.sync_copy(src_hbm.at[w], srcall)
  pltpu.sync_copy(dst_hbm.at[w], dstall)
  pltpu.sync_copy(ef_hbm.at[w], efall)
  pltpu.sync_copy(sout_hbm, sout_v)
  pltpu.sync_copy(sin_hbm, sin_v)

  def wrow(v, _):
    for j in range(CHUNK // 16):
      sl = pl.ds(j * 16, 16)
      so = plsc.load_gather(sout_v, [srcall[v, sl]])
      si = plsc.load_gather(sin_v, [dstall[v, sl]])
      efall[v, sl] = efall[v, sl] * so * si
    return 0
  lax.fori_loop(0, CPW, wrow, 0)

  pltpu.sync_copy(efall, w_hbm.at[w])


def _wcalc(src2d, dst2d, ef2d, s_out, s_in):
  fn = pl.kernel(
      _wcalc_body,
      out_type=jax.ShapeDtypeStruct((NW, CPW, CHUNK), jnp.float32),
      scratch_types=[
          pltpu.VMEM((CPW, CHUNK), jnp.int32),
          pltpu.VMEM((CPW, CHUNK), jnp.int32),
          pltpu.VMEM((CPW, CHUNK), jnp.float32),
          pltpu.VMEM((NPAD,), jnp.float32),
          pltpu.VMEM((NPAD,), jnp.float32),
      ],
      **_SC_MESH,
  )
  return fn(src2d, dst2d, ef2d, s_out, s_in)


def _conv_body(ha_hbm, hb_hbm, src_hbm, dst_hbm, w_hbm, out_hbm,
               srcall, dstall, wall, rows0, rows1, rows2, rows3,
               agg_sh, g0, g1, g2, g3, s0, s1, s2, s3):
  rows = (rows0, rows1, rows2, rows3)
  gsem = (g0, g1, g2, g3)
  ssem = (s0, s1, s2, s3)
  cid = lax.axis_index("c")
  sid = lax.axis_index("s")
  w = cid * NS + sid

  pltpu.sync_copy(src_hbm.at[w], srcall)
  pltpu.sync_copy(dst_hbm.at[w], dstall)
  pltpu.sync_copy(w_hbm.at[w], wall)

  for f in range(2):
    h_hbm = (ha_hbm, hb_hbm)[f]

    def zrow(i, _):
      for j in range(DH // 16):
        rows0[i, pl.ds(j * 16, 16)] = jnp.zeros((16,), jnp.float32)
      return 0
    lax.fori_loop(0, CHUNK, zrow, 0)
    for r in range(RPW // CHUNK):
      pltpu.sync_copy(rows0, agg_sh.at[pl.ds(sid * RPW + r * CHUNK, CHUNK)])
    plsc.subcore_barrier()

    for b in range(NBUF):
      pltpu.async_copy(h_hbm.at[srcall.at[b]], rows[b], gsem[b])

    def tbody(t, _):
      for b in range(NBUF):
        v = NBUF * t + b
        rb = rows[b]
        pltpu.make_async_copy(h_hbm.at[srcall.at[v]], rb, gsem[b]).wait()

        def erow(e2, _):
          for u in range(2):
            e = e2 * 2 + u
            we = plsc.load_gather(
                wall, [jnp.full((16,), v, jnp.int32),
                       jnp.full((16,), e, jnp.int32)])
            for j in range(DH // 16):
              sl = pl.ds(j * 16, 16)
              rb[e, sl] = rb[e, sl] * we
          return 0
        lax.fori_loop(0, CHUNK // 2, erow, 0)

        pltpu.async_copy(rb, agg_sh.at[dstall.at[v]], ssem[b], add=True)

        bp = (b + NBUF - 1) % NBUF
        cond = (t >= 1) if b == 0 else (t < T_ITERS - 1)

        @pl.when(cond)
        def _():
          pltpu.make_async_copy(
              rows[bp], agg_sh.at[dstall.at[v - 1]], ssem[bp]).wait()
          pltpu.async_copy(
              h_hbm.at[srcall.at[v + NBUF - 1]], rows[bp], gsem[bp])
      return 0
    lax.fori_loop(0, T_ITERS, tbody, 0)

    for b in range(NBUF):
      pltpu.make_async_copy(
          rows[b], agg_sh.at[dstall.at[CPW - NBUF + b]], ssem[b]).wait()

    plsc.subcore_barrier()
    sl = pl.ds(sid * RPW, RPW)
    pltpu.sync_copy(agg_sh.at[sl], out_hbm.at[f, cid, sl])
    plsc.subcore_barrier()


def _conv(ha, hb, src2d, dst2d, w2d):
  fn = pl.kernel(
      _conv_body,
      out_type=jax.ShapeDtypeStruct((2, NC, NPAD, DH), jnp.float32),
      mesh=plsc.VectorSubcoreMesh(core_axis_name="c", subcore_axis_name="s"),
      compiler_params=pltpu.CompilerParams(
          needs_layout_passes=False, use_tc_tiling_on_sc=False),
      scratch_types=[
          pltpu.VMEM((CPW, CHUNK), jnp.int32),
          pltpu.VMEM((CPW, CHUNK), jnp.int32),
          pltpu.VMEM((CPW, CHUNK), jnp.float32),
          pltpu.VMEM((CHUNK, DH), jnp.float32),
          pltpu.VMEM((CHUNK, DH), jnp.float32),
          pltpu.VMEM((CHUNK, DH), jnp.float32),
          pltpu.VMEM((CHUNK, DH), jnp.float32),
          pltpu.VMEM_SHARED((NPAD, DH), jnp.float32),
          pltpu.SemaphoreType.DMA,
          pltpu.SemaphoreType.DMA,
          pltpu.SemaphoreType.DMA,
          pltpu.SemaphoreType.DMA,
          pltpu.SemaphoreType.DMA,
          pltpu.SemaphoreType.DMA,
          pltpu.SemaphoreType.DMA,
          pltpu.SemaphoreType.DMA,
      ],
  )
  return fn(ha, hb, src2d, dst2d, w2d)


def _mm_body_split(p_ref, w_ref, b_ref, o_ref):
  p = p_ref[...]
  wm = w_ref[...]
  res = (jax.lax.dot_general(
      p[0, 0] + p[0, 1], wm[:DH], (((1,), (0,)), ((), ())),
      preferred_element_type=jnp.float32,
      precision=lax.Precision.HIGHEST) +
         jax.lax.dot_general(
      p[1, 0] + p[1, 1], wm[DH:], (((1,), (0,)), ((), ())),
      preferred_element_type=jnp.float32,
      precision=lax.Precision.HIGHEST) + b_ref[...])
  o_ref[0] = res[:, :DH]
  o_ref[1] = res[:, DH:]


def _mm_body_flat(p_ref, w_ref, b_ref, o_ref):
  p = p_ref[...]
  wm = w_ref[...]
  o_ref[...] = (jax.lax.dot_general(
      p[0, 0] + p[0, 1], wm[:DH], (((1,), (0,)), ((), ())),
      preferred_element_type=jnp.float32,
      precision=lax.Precision.HIGHEST) +
                jax.lax.dot_general(
      p[1, 0] + p[1, 1], wm[DH:], (((1,), (0,)), ((), ())),
      preferred_element_type=jnp.float32,
      precision=lax.Precision.HIGHEST) + b_ref[...])


def _dense(parts, W, b, nrows, block, split_out):
  grid = nrows // block
  if split_out:
    body = _mm_body_split
    out_shape = jax.ShapeDtypeStruct((2, nrows, DH), jnp.float32)
    out_specs = pl.BlockSpec((2, block, DH), lambda i: (0, i, 0))
  else:
    body = _mm_body_flat
    out_shape = jax.ShapeDtypeStruct((nrows, D), jnp.float32)
    out_specs = pl.BlockSpec((block, D), lambda i: (i, 0))
  return pl.pallas_call(
      body,
      grid=(grid,),
      in_specs=[
          pl.BlockSpec((2, NC, block, DH), lambda i: (0, 0, i, 0)),
          pl.BlockSpec((D, D), lambda i: (0, 0)),
          pl.BlockSpec((1, D), lambda i: (0, 0)),
      ],
      out_specs=out_specs,
      out_shape=out_shape,
  )(parts, W, b.reshape(1, D))


def kernel(in_feat, edge_index, e_feat, W0, b0, W1, b1):
  src = edge_index[0].astype(jnp.int32)
  dst = edge_index[1].astype(jnp.int32)
  pad_idx = jnp.full((EPAD - E,), NPAD - 1, jnp.int32)
  src2d = jnp.concatenate([src, pad_idx]).reshape(NW, CPW, CHUNK)
  dst2d = jnp.concatenate([dst, pad_idx]).reshape(NW, CPW, CHUNK)
  ef2d = jnp.concatenate(
      [e_feat, jnp.zeros((EPAD - E,), jnp.float32)]).reshape(NW, CPW, CHUNK)
  h0 = jnp.pad(in_feat, ((0, NPAD - N), (0, 0)))
  h0a, h0b = h0[:, :DH], h0[:, DH:]

  deg_parts = _degrees(src2d, dst2d)
  s = _scales(deg_parts)
  w2d = _wcalc(src2d, dst2d, ef2d, s[0], s[1])

  parts1 = _conv(h0a, h0b, src2d, dst2d, w2d)
  h1 = _dense(parts1, W0, b0, NPAD, 640, split_out=True)
  parts2 = _conv(h1[0], h1[1], src2d, dst2d, w2d)
  return _dense(parts2, W1, b1, N, 400, split_out=False)
